# Initial kernel scaffold; baseline (speedup 1.0000x reference)
import jax
import jax.numpy as jnp
from jax.experimental import pallas as pl


def kernel(x, edge_index, batch, W_init, b_init, W_rel, b_rel, W_root, gamma, beta):
    raise NotImplementedError("write your pallas kernel here")


# speedup vs baseline: 6.4212x; 6.4212x over previous
import functools

import jax
import jax.numpy as jnp
from jax import lax
from jax.experimental import pallas as pl
from jax.experimental.pallas import tpu as pltpu
from jax.experimental.pallas import tpu_sc as plsc

N = 10000
E = 320000
D = 128
G = 64
L = 3

NC = 2
NS = 16
NW = NC * NS
NPAD = 10240
RPS = NPAD // NS
WIN = 128
NWIN = E // WIN
WPW = -(-NWIN // NW)

_mesh = plsc.VectorSubcoreMesh(core_axis_name="c", subcore_axis_name="s")


@functools.partial(
    pl.kernel,
    out_type=jax.ShapeDtypeStruct((NC, NPAD, D), jnp.float32),
    mesh=_mesh,
    scratch_types=[
        pltpu.VMEM_SHARED((NPAD, D), jnp.float32),
        pltpu.VMEM((WIN,), jnp.int32),
        pltpu.VMEM((WIN,), jnp.int32),
        pltpu.VMEM((WIN, D), jnp.float32),
    ],
)
def _sc_agg_kernel(h_hbm, e_hbm, z_hbm, out_hbm, acc, src_v, dst_v, rows_v):
    c = lax.axis_index("c")
    s = lax.axis_index("s")
    wid = s * NC + c

    pltpu.sync_copy(z_hbm, acc.at[pl.ds(s * RPS, RPS)])
    plsc.subcore_barrier()

    @pl.loop(0, WPW)
    def _(j):
        w = wid + NW * j

        @pl.when(w < NWIN)
        def _():
            off = w * WIN
            pltpu.sync_copy(e_hbm.at[0, pl.ds(off, WIN)], src_v)
            pltpu.sync_copy(e_hbm.at[1, pl.ds(off, WIN)], dst_v)
            pltpu.sync_copy(h_hbm.at[src_v], rows_v)
            pltpu.sync_copy(rows_v, acc.at[dst_v], add=True)

    plsc.subcore_barrier()
    pltpu.sync_copy(acc.at[pl.ds(s * RPS, RPS)],
                    out_hbm.at[c, pl.ds(s * RPS, RPS)])


def _sc_agg(h, edge_index, zeros):
    return _sc_agg_kernel(h, edge_index, zeros)


def _dot_t(a, w):
    return lax.dot_general(a, w, (((1,), (1,)), ((), ())),
                           preferred_element_type=jnp.float32)


def _tc_init_body(x_ref, w_ref, b_ref, o_ref):
    o_ref[...] = _dot_t(x_ref[...], w_ref[...]) + b_ref[...]


def _tc_init(x, W_init, b2):
    return pl.pallas_call(
        _tc_init_body,
        out_shape=jax.ShapeDtypeStruct((N, D), jnp.float32),
    )(x, W_init, b2)


def _tc_layer_body(h_ref, p_ref, wr_ref, br_ref, wt_ref, g_ref, b_ref, o_ref):
    agg = p_ref[0, :N, :] + p_ref[1, :N, :]
    h = h_ref[...]
    t = h + _dot_t(agg, wr_ref[...]) + br_ref[...] + _dot_t(h, wt_ref[...])
    m = jnp.mean(t, axis=0, keepdims=True)
    v = jnp.mean((t - m) ** 2, axis=0, keepdims=True)
    t = (t - m) / jnp.sqrt(v + 1e-5) * g_ref[...] + b_ref[...]
    o_ref[...] = jnp.maximum(t, 0.0)


def _tc_layer(h, parts, Wr, br2, Wt, g2, b2):
    return pl.pallas_call(
        _tc_layer_body,
        out_shape=jax.ShapeDtypeStruct((N, D), jnp.float32),
    )(h, parts, Wr, br2, Wt, g2, b2)


def _tc_final_body(h_ref, p_ref, wr_ref, br_ref, wt_ref, batch_ref, o_ref):
    agg = p_ref[0, :N, :] + p_ref[1, :N, :]
    t = _dot_t(agg, wr_ref[...]) + br_ref[...] + _dot_t(h_ref[...], wt_ref[...])
    seg = lax.broadcasted_iota(jnp.int32, (G, N), 0)
    mask = (seg == batch_ref[...]).astype(jnp.float32)
    o_ref[...] = lax.dot_general(mask, t, (((1,), (0,)), ((), ())),
                                 preferred_element_type=jnp.float32)


def _tc_final(h, parts, Wr, br2, Wt, batch2):
    return pl.pallas_call(
        _tc_final_body,
        out_shape=jax.ShapeDtypeStruct((G, D), jnp.float32),
    )(h, parts, Wr, br2, Wt, batch2)


def kernel(x, edge_index, batch, W_init, b_init, W_rel, b_rel, W_root, gamma, beta):
    zeros = jnp.zeros((RPS, D), jnp.float32)
    batch2 = batch.reshape(1, N)
    h = _tc_init(x, W_init, b_init.reshape(1, D))
    for i in range(L - 1):
        parts = _sc_agg(h, edge_index, zeros)
        h = _tc_layer(h, parts, W_rel[i], b_rel[i].reshape(1, D),
                      W_root[i], gamma[i].reshape(1, D), beta[i].reshape(1, D))
    parts = _sc_agg(h, edge_index, zeros)
    return _tc_final(h, parts, W_rel[L - 1], b_rel[L - 1].reshape(1, D),
                     W_root[L - 1], batch2)
